# baseline (speedup 1.0000x reference)
import jax
import jax.numpy as jnp
from jax import lax
from jax.experimental import pallas as pl
from jax.experimental.pallas import tpu as pltpu
from jax.experimental.pallas import tpu_sc as plsc

EMBED = 64
SEQ = 200
BATCH = 4096
NW = 32
BPW = BATCH // NW
NB = 4
NT = 2
TOKW = 128
TSTR = 129
LANES = 16


def _body(xt_hbm, tok_hbm, pos_hbm, out_hbm, idx_v, pos_v, rows_v, t_v,
          gsem0, gsem1, gsem2, gsem3, osem0, osem1):
    gsems = (gsem0, gsem1, gsem2, gsem3)
    osems = (osem0, osem1)
    wid = lax.axis_index("s") * 2 + lax.axis_index("c")

    pltpu.sync_copy(xt_hbm.at[:, pl.ds(wid * BPW, BPW)], idx_v)
    pltpu.sync_copy(pos_hbm, pos_v)

    iota = lax.iota(jnp.int32, LANES)

    def fire_gather(s, b):
        pltpu.async_copy(tok_hbm.at[idx_v.at[s]],
                         rows_v.at[pl.ds(b * BPW, BPW)], gsems[b])

    def drain_gather(s, b):
        pltpu.make_async_copy(tok_hbm.at[idx_v.at[s]],
                              rows_v.at[pl.ds(b * BPW, BPW)],
                              gsems[b]).wait()

    def fire_out(s, tb):
        for fh in range(EMBED // 8):
            pltpu.async_copy(
                t_v.at[pl.ds(tb * EMBED + fh * 8, 8), pl.ds(0, BPW)],
                out_hbm.at[s, fh, wid], osems[tb])

    def drain_out(s, tb):
        for fh in range(EMBED // 8):
            pltpu.make_async_copy(
                t_v.at[pl.ds(tb * EMBED + fh * 8, 8), pl.ds(0, BPW)],
                out_hbm.at[s, fh, wid],
                osems[tb]).wait()

    def transpose_add(s, b, tb):
        base = b * BPW
        ps = [pos_v[s, pl.ds(c * LANES, LANES)]
              for c in range(EMBED // LANES)]
        rvecs = [iota + (c * LANES + tb * EMBED)
                 for c in range(EMBED // LANES)]
        unroll = 16

        def tok_block(k, carry):
            for u in range(unroll):
                bb = k * unroll + u
                cvec = jnp.full((LANES,), bb, jnp.int32)
                for c in range(EMBED // LANES):
                    v = rows_v[base + bb, pl.ds(c * LANES, LANES)] + ps[c]
                    plsc.store_scatter(t_v, [rvecs[c], cvec], v)
            return carry

        lax.fori_loop(0, BPW // unroll, tok_block, 0)

    for b in range(NB - 1):
        fire_gather(b, b)

    def outer(i, carry):
        for b in range(NB):
            s = i * NB + b
            tb = b % NT
            drain_gather(s, b)

            @pl.when(s + NB - 1 < SEQ)
            def _ahead():
                fire_gather(s + NB - 1, (b - 1) % NB)

            @pl.when(s >= NT)
            def _reclaim():
                drain_out(s - NT, tb)

            transpose_add(s, b, tb)
            fire_out(s, tb)
        return carry

    lax.fori_loop(0, SEQ // NB, outer, 0)
    for tb in range(NT):
        s_last = max(s for s in range(SEQ) if s % NT == tb)
        drain_out(s_last, tb)


def _impl(xt, tok, pos):
    mesh = plsc.VectorSubcoreMesh(core_axis_name="c", subcore_axis_name="s")
    f = pl.kernel(
        _body,
        mesh=mesh,
        out_type=jax.ShapeDtypeStruct(
            (SEQ, EMBED // 8, BATCH // 128, 8, 128), jnp.float32),
        scratch_types=[
            pltpu.VMEM((SEQ, BPW), jnp.int32),
            pltpu.VMEM((SEQ, EMBED), jnp.float32),
            pltpu.VMEM((NB * BPW, TOKW), jnp.float32),
            pltpu.VMEM((NT * EMBED, TSTR), jnp.float32),
        ] + [pltpu.SemaphoreType.DMA] * (NB + NT),
        compiler_params=pltpu.CompilerParams(use_tc_tiling_on_sc=False,
                                             needs_layout_passes=False),
    )
    return f(xt, tok, pos)


def kernel(x, token_table, pos_table):
    batch, seq = x.shape
    xt = jnp.swapaxes(x.astype(jnp.int32), 0, 1)
    tok128 = jnp.pad(token_table, ((0, 0), (0, TOKW - EMBED)))
    out5 = _impl(xt, tok128, pos_table)
    return out5.transpose(2, 4, 0, 1, 3).reshape(batch, seq, EMBED)


# speedup vs baseline: 1.0021x; 1.0021x over previous
import jax
import jax.numpy as jnp
from jax import lax
from jax.experimental import pallas as pl
from jax.experimental.pallas import tpu as pltpu
from jax.experimental.pallas import tpu_sc as plsc

EMBED = 64
SEQ = 200
BATCH = 4096
NW = 32
BPW = BATCH // NW
NB = 4
NT = 2
TOKW = 128
TSTR = 129
LANES = 16


def _body(xt_hbm, tok_hbm, pos_hbm, out_hbm, idx_v, pos_v, rows_v, t_v,
          gsem0, gsem1, gsem2, gsem3, osem0, osem1):
    gsems = (gsem0, gsem1, gsem2, gsem3)
    osems = (osem0, osem1)
    wid = lax.axis_index("s") * 2 + lax.axis_index("c")

    pltpu.sync_copy(xt_hbm.at[:, pl.ds(wid * BPW, BPW)], idx_v)
    pltpu.sync_copy(pos_hbm, pos_v)

    iota = lax.iota(jnp.int32, LANES)

    def fire_gather(s, b):
        pltpu.async_copy(tok_hbm.at[idx_v.at[s]],
                         rows_v.at[pl.ds(b * BPW, BPW)], gsems[b])

    def drain_gather(s, b):
        pltpu.make_async_copy(tok_hbm.at[idx_v.at[s]],
                              rows_v.at[pl.ds(b * BPW, BPW)],
                              gsems[b]).wait()

    def fire_out(s, tb):
        for fh in range(EMBED // 8):
            pltpu.async_copy(
                t_v.at[pl.ds(tb * EMBED + fh * 8, 8), pl.ds(0, BPW)],
                out_hbm.at[s, fh, wid], osems[tb])

    def drain_out(s, tb):
        for fh in range(EMBED // 8):
            pltpu.make_async_copy(
                t_v.at[pl.ds(tb * EMBED + fh * 8, 8), pl.ds(0, BPW)],
                out_hbm.at[s, fh, wid],
                osems[tb]).wait()

    def transpose_add(s, b, tb):
        base = b * BPW
        ps = [pos_v[s, pl.ds(c * LANES, LANES)]
              for c in range(EMBED // LANES)]
        rvecs = [iota + (c * LANES + tb * EMBED)
                 for c in range(EMBED // LANES)]
        unroll = 16

        def tok_block(k, carry):
            for u in range(unroll):
                bb = k * unroll + u
                cvec = jnp.full((LANES,), bb, jnp.int32)
                for c in range(EMBED // LANES):
                    v = rows_v[base + bb, pl.ds(c * LANES, LANES)] + ps[c]
                    plsc.store_scatter(t_v, [rvecs[c], cvec], v)
            return carry

        lax.fori_loop(0, BPW // unroll, tok_block, 0)

    for b in range(NB - 1):
        fire_gather(b, b)

    def outer(i, carry):
        for b in range(NB):
            s = i * NB + b
            tb = b % NT
            drain_gather(s, b)

            @pl.when(s + NB - 1 < SEQ)
            def _ahead():
                fire_gather(s + NB - 1, (b - 1) % NB)

            @pl.when(s >= NT)
            def _reclaim():
                drain_out(s - NT, tb)

            transpose_add(s, b, tb)
            fire_out(s, tb)
        return carry

    lax.fori_loop(0, SEQ // NB, outer, 0)
    for tb in range(NT):
        s_last = max(s for s in range(SEQ) if s % NT == tb)
        drain_out(s_last, tb)


def _impl(xt, tok, pos):
    mesh = plsc.VectorSubcoreMesh(core_axis_name="c", subcore_axis_name="s")
    f = pl.kernel(
        _body,
        mesh=mesh,
        out_type=jax.ShapeDtypeStruct(
            (SEQ, EMBED // 8, BATCH // 128, 8, 128), jnp.float32),
        scratch_types=[
            pltpu.VMEM((SEQ, BPW), jnp.int32),
            pltpu.VMEM((SEQ, EMBED), jnp.float32),
            pltpu.VMEM((NB * BPW, EMBED), jnp.float32),
            pltpu.VMEM((NT * EMBED, TSTR), jnp.float32),
        ] + [pltpu.SemaphoreType.DMA] * (NB + NT),
        compiler_params=pltpu.CompilerParams(use_tc_tiling_on_sc=False,
                                             needs_layout_passes=False),
    )
    return f(xt, tok, pos)


def kernel(x, token_table, pos_table):
    batch, seq = x.shape
    xt = jnp.swapaxes(x.astype(jnp.int32) * 2, 0, 1)
    tok2 = jnp.pad(token_table, ((0, 0), (0, TOKW - EMBED)))
    tok2 = tok2.reshape(2 * tok2.shape[0], EMBED)
    out5 = _impl(xt, tok2, pos_table)
    return out5.transpose(2, 4, 0, 1, 3).reshape(batch, seq, EMBED)
